# baseline (speedup 1.0000x reference)
import functools

import jax
import jax.numpy as jnp
from jax import lax
from jax.experimental import pallas as pl
from jax.experimental.pallas import tpu as pltpu
from jax.experimental.pallas import tpu_sc as plsc

_NC = 2
_NS = 16
_NW = _NC * _NS
_R = 512
_G = _R // 16


@functools.partial(jax.jit, static_argnums=(3, 4))
def _sc_proj(inputs, low16, invw16, n, c):
    cols_per_w = n // _NW
    n_chunks = cols_per_w // _R

    mesh = plsc.VectorSubcoreMesh(core_axis_name="c", subcore_axis_name="s")

    @functools.partial(
        pl.kernel,
        mesh=mesh,
        out_type=jax.ShapeDtypeStruct((c, n), jnp.float32),
        compiler_params=pltpu.CompilerParams(needs_layout_passes=False),
        scratch_types=[
            pltpu.VMEM((cols_per_w,), jnp.float32),
            pltpu.VMEM((c, _R), jnp.float32),
            pltpu.VMEM((c, _R), jnp.float32),
            pltpu.VMEM((_R,), jnp.int32),
            pltpu.VMEM((_R,), jnp.int32),
            pltpu.VMEM((16,), jnp.float32),
            pltpu.VMEM((16,), jnp.float32),
            pltpu.SemaphoreType.DMA,
            pltpu.SemaphoreType.DMA,
            pltpu.SemaphoreType.DMA,
        ],
    )
    def k(in_hbm, low_hbm, invw_hbm, out_hbm,
          in_v, out_a, out_b, idx_a, idx_b, low_v, invw_v,
          sem_a, sem_b, sem_in):
        wid = lax.axis_index("s") * _NC + lax.axis_index("c")
        col0 = wid * cols_per_w

        in_copy = pltpu.async_copy(
            in_hbm.at[pl.ds(col0, cols_per_w)], in_v, sem_in)
        pltpu.sync_copy(low_hbm, low_v)
        pltpu.sync_copy(invw_hbm, invw_v)
        low = low_v[...]
        invw = invw_v[...]
        lane = lax.iota(jnp.int32, 16)
        zeros16 = jnp.zeros((16,), jnp.float32)

        outs = (out_a, out_b)
        idxs = (idx_a, idx_b)
        sems = (sem_a, sem_b)

        def memset_body(j, carry):
            f = j * 16 + lane
            r = lax.shift_right_logical(f, _R.bit_length() - 1)
            q = lax.bitwise_and(f, _R - 1)
            plsc.store_scatter(out_a, [r, q], zeros16)
            plsc.store_scatter(out_b, [r, q], zeros16)
            return carry
        lax.fori_loop(0, c * _R // 16, memset_body, 0)
        in_copy.wait()

        def compute_chunk(chunk, b):
            cbase = col0 + chunk * _R

            def grp(j, carry):
                x = in_v[pl.ds(chunk * _R + j * 16, 16)]
                u = (x - low) * invw
                i = u.astype(jnp.int32)
                p = u - i.astype(jnp.float32)
                cols = j * 16 + lane
                plsc.store_scatter(outs[b], [i, cols], 1.0 - p)
                plsc.store_scatter(outs[b], [i + 1, cols], p)
                idxs[b][pl.ds(j * 16, 16)] = i
                return carry
            lax.fori_loop(0, _G, grp, 0)
            pltpu.async_copy(outs[b], out_hbm.at[:, pl.ds(cbase, _R)], sems[b])

        def wait_and_rezero(b):
            pltpu.make_async_copy(
                outs[b], out_hbm.at[:, pl.ds(0, _R)], sems[b]).wait()

            def rz(j, carry):
                i = idxs[b][pl.ds(j * 16, 16)]
                cols = j * 16 + lane
                plsc.store_scatter(outs[b], [i, cols], zeros16)
                plsc.store_scatter(outs[b], [i + 1, cols], zeros16)
                return carry
            lax.fori_loop(0, _G, rz, 0)

        compute_chunk(0, 0)
        compute_chunk(1, 1)

        def main_body(jj, carry):
            for b in range(2):
                wait_and_rezero(b)
                compute_chunk(2 * jj + b, b)
            return carry
        lax.fori_loop(1, n_chunks // 2, main_body, 0)

        pltpu.make_async_copy(out_a, out_hbm.at[:, pl.ds(0, _R)], sem_a).wait()
        pltpu.make_async_copy(out_b, out_hbm.at[:, pl.ds(0, _R)], sem_b).wait()

    return k(inputs, low16, invw16)


def kernel(inputs, centers):
    n = inputs.shape[0]
    c = centers.shape[0]
    low16 = jnp.broadcast_to(centers[0], (16,)).astype(jnp.float32)
    invw16 = jnp.broadcast_to(
        1.0 / (centers[1] - centers[0]), (16,)).astype(jnp.float32)
    return _sc_proj(inputs, low16, invw16, n, c).T


# speedup vs baseline: 1.0322x; 1.0322x over previous
import functools

import jax
import jax.numpy as jnp
from jax import lax
from jax.experimental import pallas as pl
from jax.experimental.pallas import tpu as pltpu
from jax.experimental.pallas import tpu_sc as plsc

_NC = 2
_NS = 16
_NW = _NC * _NS
_R = 256
_NB = 4
_G = _R // 16


@functools.partial(jax.jit, static_argnums=(3, 4))
def _sc_proj(inputs, low16, invw16, n, c):
    cols_per_w = n // _NW
    n_chunks = cols_per_w // _R

    mesh = plsc.VectorSubcoreMesh(core_axis_name="c", subcore_axis_name="s")

    @functools.partial(
        pl.kernel,
        mesh=mesh,
        out_type=jax.ShapeDtypeStruct((c, n), jnp.float32),
        compiler_params=pltpu.CompilerParams(needs_layout_passes=False),
        scratch_types=(
            [pltpu.VMEM((cols_per_w,), jnp.float32)]
            + [pltpu.VMEM((c, _R), jnp.float32)] * _NB
            + [pltpu.VMEM((_R,), jnp.int32)] * _NB
            + [
                pltpu.VMEM((16,), jnp.float32),
                pltpu.VMEM((16,), jnp.float32),
            ]
            + [pltpu.SemaphoreType.DMA] * (_NB + 1)
        ),
    )
    def k(in_hbm, low_hbm, invw_hbm, out_hbm, in_v, *rest):
        outs = rest[:_NB]
        idxs = rest[_NB:2 * _NB]
        low_v, invw_v = rest[2 * _NB:2 * _NB + 2]
        sems = rest[2 * _NB + 2:3 * _NB + 2]
        sem_in = rest[3 * _NB + 2]
        wid = lax.axis_index("s") * _NC + lax.axis_index("c")
        col0 = wid * cols_per_w

        in_copy = pltpu.async_copy(
            in_hbm.at[pl.ds(col0, cols_per_w)], in_v, sem_in)
        pltpu.sync_copy(low_hbm, low_v)
        pltpu.sync_copy(invw_hbm, invw_v)
        low = low_v[...]
        invw = invw_v[...]
        lane = lax.iota(jnp.int32, 16)
        zeros16 = jnp.zeros((16,), jnp.float32)

        def memset_body(j, carry):
            f = j * 16 + lane
            r = lax.shift_right_logical(f, _R.bit_length() - 1)
            q = lax.bitwise_and(f, _R - 1)
            for b in range(_NB):
                plsc.store_scatter(outs[b], [r, q], zeros16)
            return carry
        lax.fori_loop(0, c * _R // 16, memset_body, 0)
        in_copy.wait()

        def compute_chunk(chunk, b):
            cbase = col0 + chunk * _R

            def grp(j, carry):
                x = in_v[pl.ds(chunk * _R + j * 16, 16)]
                u = (x - low) * invw
                i = u.astype(jnp.int32)
                p = u - i.astype(jnp.float32)
                cols = j * 16 + lane
                plsc.store_scatter(outs[b], [i, cols], 1.0 - p)
                plsc.store_scatter(outs[b], [i + 1, cols], p)
                idxs[b][pl.ds(j * 16, 16)] = i
                return carry
            lax.fori_loop(0, _G, grp, 0)
            pltpu.async_copy(outs[b], out_hbm.at[:, pl.ds(cbase, _R)], sems[b])

        def wait_and_rezero(b):
            pltpu.make_async_copy(
                outs[b], out_hbm.at[:, pl.ds(0, _R)], sems[b]).wait()

            def rz(j, carry):
                i = idxs[b][pl.ds(j * 16, 16)]
                cols = j * 16 + lane
                plsc.store_scatter(outs[b], [i, cols], zeros16)
                plsc.store_scatter(outs[b], [i + 1, cols], zeros16)
                return carry
            lax.fori_loop(0, _G, rz, 0)

        for b in range(_NB):
            compute_chunk(b, b)

        def main_body(jj, carry):
            for b in range(_NB):
                wait_and_rezero(b)
                compute_chunk(_NB * jj + b, b)
            return carry
        lax.fori_loop(1, n_chunks // _NB, main_body, 0)

        for b in range(_NB):
            pltpu.make_async_copy(
                outs[b], out_hbm.at[:, pl.ds(0, _R)], sems[b]).wait()

    return k(inputs, low16, invw16)


def kernel(inputs, centers):
    n = inputs.shape[0]
    c = centers.shape[0]
    low16 = jnp.broadcast_to(centers[0], (16,)).astype(jnp.float32)
    invw16 = jnp.broadcast_to(
        1.0 / (centers[1] - centers[0]), (16,)).astype(jnp.float32)
    return _sc_proj(inputs, low16, invw16, n, c).T


# speedup vs baseline: 1.0707x; 1.0372x over previous
import functools

import jax
import jax.numpy as jnp
from jax import lax
from jax.experimental import pallas as pl
from jax.experimental.pallas import tpu as pltpu
from jax.experimental.pallas import tpu_sc as plsc

_NC = 2
_NS = 16
_NW = _NC * _NS
_R = 256
_G = _R // 16


@functools.partial(jax.jit, static_argnums=(2, 3))
def _sc_proj(inputs, centers, n, c):
    cols_per_w = n // _NW
    n_chunks = cols_per_w // _R

    mesh = plsc.VectorSubcoreMesh(core_axis_name="c", subcore_axis_name="s")

    @functools.partial(
        pl.kernel,
        mesh=mesh,
        out_type=jax.ShapeDtypeStruct((c, n), jnp.float32),
        compiler_params=pltpu.CompilerParams(needs_layout_passes=False),
        scratch_types=[
            pltpu.VMEM((cols_per_w,), jnp.float32),
            pltpu.VMEM((c, _R), jnp.float32),
            pltpu.VMEM((c, _R), jnp.float32),
            pltpu.VMEM((_R,), jnp.int32),
            pltpu.VMEM((_R,), jnp.int32),
            pltpu.VMEM((16,), jnp.float32),
            pltpu.SemaphoreType.DMA,
            pltpu.SemaphoreType.DMA,
            pltpu.SemaphoreType.DMA,
        ],
    )
    def k(in_hbm, cent_hbm, out_hbm,
          in_v, out_a, out_b, idx_a, idx_b, cent_v,
          sem_a, sem_b, sem_in):
        wid = lax.axis_index("s") * _NC + lax.axis_index("c")
        col0 = wid * cols_per_w

        in_copy = pltpu.async_copy(
            in_hbm.at[pl.ds(col0, cols_per_w)], in_v, sem_in)
        pltpu.sync_copy(cent_hbm.at[pl.ds(0, 16)], cent_v)
        lane = lax.iota(jnp.int32, 16)
        zeros16 = jnp.zeros((16,), jnp.float32)
        zeros16i = jnp.zeros((16,), jnp.int32)
        c0 = plsc.load_gather(cent_v, [zeros16i])
        c1 = plsc.load_gather(cent_v, [zeros16i + 1])
        low = c0
        invw = 1.0 / (c1 - c0)

        outs = (out_a, out_b)
        idxs = (idx_a, idx_b)
        sems = (sem_a, sem_b)

        def memset_body(j, carry):
            f = j * 16 + lane
            r = lax.shift_right_logical(f, _R.bit_length() - 1)
            q = lax.bitwise_and(f, _R - 1)
            plsc.store_scatter(out_a, [r, q], zeros16)
            plsc.store_scatter(out_b, [r, q], zeros16)
            return carry
        lax.fori_loop(0, c * _R // 16, memset_body, 0)
        in_copy.wait()

        def compute_chunk(chunk, b, rezero):
            cbase = col0 + chunk * _R

            def grp(j, carry):
                cols = j * 16 + lane
                if rezero:
                    oi = idxs[b][pl.ds(j * 16, 16)]
                    plsc.store_scatter(outs[b], [oi, cols], zeros16)
                    plsc.store_scatter(outs[b], [oi + 1, cols], zeros16)
                x = in_v[pl.ds(chunk * _R + j * 16, 16)]
                u = (x - low) * invw
                i = u.astype(jnp.int32)
                p = u - i.astype(jnp.float32)
                plsc.store_scatter(outs[b], [i, cols], 1.0 - p)
                plsc.store_scatter(outs[b], [i + 1, cols], p)
                idxs[b][pl.ds(j * 16, 16)] = i
                return carry
            lax.fori_loop(0, _G, grp, 0)
            pltpu.async_copy(outs[b], out_hbm.at[:, pl.ds(cbase, _R)], sems[b])

        compute_chunk(0, 0, False)
        compute_chunk(1, 1, False)

        def main_body(jj, carry):
            for b in range(2):
                pltpu.make_async_copy(
                    outs[b], out_hbm.at[:, pl.ds(0, _R)], sems[b]).wait()
                compute_chunk(2 * jj + b, b, True)
            return carry
        lax.fori_loop(1, n_chunks // 2, main_body, 0)

        pltpu.make_async_copy(out_a, out_hbm.at[:, pl.ds(0, _R)], sem_a).wait()
        pltpu.make_async_copy(out_b, out_hbm.at[:, pl.ds(0, _R)], sem_b).wait()

    return k(inputs, centers)


def kernel(inputs, centers):
    n = inputs.shape[0]
    c = centers.shape[0]
    return _sc_proj(inputs, centers.astype(jnp.float32), n, c).T
